# baseline (speedup 1.0000x reference)
import functools

import jax
import jax.numpy as jnp
from jax import lax
from jax.experimental import pallas as pl
from jax.experimental.pallas import tpu as pltpu
from jax.experimental.pallas import tpu_sc as plsc

_B, _F, _FW, _L, _V, _D = 4096, 2, 1, 20, 100000, 64
_NF = _F + _FW
_NW = 32
_NP = _NF * _D
_PPW = _NP // _NW
_CB = 128
_NCH = _B // _CB
_NG = _CB // 16


def _body(ebc_t, w_t, idx_hbm, wts_hbm, out_hbm,
          row_v, idx_v0, idx_v1, wts_v0, wts_v1, out_row,
          sem_a, sem_b):
    idx_bufs = (idx_v0, idx_v1)
    wts_bufs = (wts_v0, wts_v1)
    slot = lax.axis_index("s")
    sc = lax.axis_index("c")
    wid = slot * 2 + sc

    sems = (sem_a, sem_b)

    def start_chunk(f, c, b, weighted):
        pltpu.async_copy(idx_hbm.at[f, c], idx_bufs[b], sems[b])
        if weighted:
            pltpu.async_copy(wts_hbm.at[c], wts_bufs[b], sems[b])

    def wait_chunk(f, c, b, weighted):
        pltpu.make_async_copy(idx_hbm.at[f, c], idx_bufs[b], sems[b]).wait()
        if weighted:
            pltpu.make_async_copy(wts_hbm.at[c], wts_bufs[b], sems[b]).wait()

    def accum_chunk(c, b, weighted):
        def group_body(g, carry):
            gl = pl.ds(pl.multiple_of(g * 16, 16), 16)
            ivs = [idx_bufs[b][l, gl] for l in range(_L)]
            vals = [plsc.load_gather(row_v, [iv]) for iv in ivs]
            if weighted:
                ws = [wts_bufs[b][l, gl] for l in range(_L)]
                vals = [v * w for v, w in zip(vals, ws)]
            while len(vals) > 1:
                nxt = [vals[i] + vals[i + 1]
                       for i in range(0, len(vals) - 1, 2)]
                if len(vals) % 2:
                    nxt.append(vals[-1])
                vals = nxt
            out_row[pl.ds(pl.multiple_of(c * _CB + g * 16, 16), 16)] = vals[0]
            return carry
        lax.fori_loop(0, _NG, group_body, 0)

    def do_pair(k):
        p_lo = 32 * k
        f = p_lo // _D
        weighted = f == _F
        d = wid + 32 * (k % 2)
        tab = w_t if weighted else ebc_t
        pltpu.sync_copy(tab.at[0 if weighted else f, d], row_v)
        start_chunk(f, 0, 0, weighted)

        def step(i, carry):
            a = 2 * i
            wait_chunk(f, a, 0, weighted)
            start_chunk(f, a + 1, 1, weighted)
            accum_chunk(a, 0, weighted)
            wait_chunk(f, a + 1, 1, weighted)

            @pl.when(i < _NCH // 2 - 1)
            def _():
                start_chunk(f, a + 2, 0, weighted)
            accum_chunk(a + 1, 1, weighted)
            return carry
        lax.fori_loop(0, _NCH // 2, step, 0)
        pltpu.sync_copy(out_row, out_hbm.at[wid + 32 * k])

    for k in range(_PPW):
        do_pair(k)


@jax.jit
def _run(ebc_t, w_t, idx_all, wts):
    mesh = plsc.VectorSubcoreMesh(core_axis_name="c", subcore_axis_name="s")
    k = functools.partial(
        pl.kernel,
        mesh=mesh,
        out_type=jax.ShapeDtypeStruct((_NP, _B), jnp.float32),
        scratch_types=[
            pltpu.VMEM((_V,), jnp.float32),
            pltpu.VMEM((_L, _CB), jnp.int32),
            pltpu.VMEM((_L, _CB), jnp.int32),
            pltpu.VMEM((_L, _CB), jnp.float32),
            pltpu.VMEM((_L, _CB), jnp.float32),
            pltpu.VMEM((_B,), jnp.float32),
            pltpu.SemaphoreType.DMA,
            pltpu.SemaphoreType.DMA,
        ],
        compiler_params=pltpu.CompilerParams(needs_layout_passes=False),
    )(_body)
    return k(ebc_t, w_t, idx_all, wts)


def kernel(features_indices, weighted_features_indices,
           weighted_features_weights, ebc_tables, weighted_tables):
    ebc_t = ebc_tables.transpose(0, 2, 1)
    w_t = weighted_tables.transpose(0, 2, 1)
    it = features_indices.astype(jnp.int32).transpose(2, 1, 0)
    f0 = it[:, 0, :].reshape(_L, _NCH, _CB).transpose(1, 0, 2)
    f1 = it[:, 1, :].reshape(_L, _NCH, _CB).transpose(1, 0, 2)
    wi = weighted_features_indices.astype(jnp.int32).transpose(2, 1, 0)
    wi = wi[:, 0, :].reshape(_L, _NCH, _CB).transpose(1, 0, 2)
    idx_all = jnp.stack([f0, f1, wi])
    wts = weighted_features_weights.transpose(2, 1, 0)
    wts = wts[:, 0, :].reshape(_L, _NCH, _CB).transpose(1, 0, 2)
    out = _run(ebc_t, w_t, idx_all, wts)
    return out.T


# speedup vs baseline: 1.0002x; 1.0002x over previous
import functools

import jax
import jax.numpy as jnp
from jax import lax
from jax.experimental import pallas as pl
from jax.experimental.pallas import tpu as pltpu
from jax.experimental.pallas import tpu_sc as plsc

_B, _F, _FW, _L, _V, _D = 4096, 2, 1, 20, 100000, 64
_NF = _F + _FW
_NW = 32
_NP = _NF * _D
_PPW = _NP // _NW
_CB = 128
_NCH = _B // _CB
_NG = _CB // 16


def _body(ebc_t, w_t, idx_hbm, wts_hbm, out_hbm,
          row_v, idx_v0, idx_v1, wts_v0, wts_v1, out_row,
          sem_a, sem_b):
    idx_bufs = (idx_v0, idx_v1)
    wts_bufs = (wts_v0, wts_v1)
    slot = lax.axis_index("s")
    sc = lax.axis_index("c")
    wid = slot * 2 + sc

    sems = (sem_a, sem_b)

    def start_chunk(f, c, b, weighted):
        pltpu.async_copy(idx_hbm.at[f, c], idx_bufs[b], sems[b])
        if weighted:
            pltpu.async_copy(wts_hbm.at[c], wts_bufs[b], sems[b])

    def wait_chunk(f, c, b, weighted):
        pltpu.make_async_copy(idx_hbm.at[f, c], idx_bufs[b], sems[b]).wait()
        if weighted:
            pltpu.make_async_copy(wts_hbm.at[c], wts_bufs[b], sems[b]).wait()

    def accum_chunk(c, b, weighted):
        def group_body(g2, carry):
            for u in range(2):
                g = 2 * g2 + u
                gl = pl.ds(pl.multiple_of(g * 16, 16), 16)
                ivs = [idx_bufs[b][l, gl] for l in range(_L)]
                vals = [plsc.load_gather(row_v, [iv]) for iv in ivs]
                if weighted:
                    ws = [wts_bufs[b][l, gl] for l in range(_L)]
                    vals = [v * w for v, w in zip(vals, ws)]
                while len(vals) > 1:
                    nxt = [vals[i] + vals[i + 1]
                           for i in range(0, len(vals) - 1, 2)]
                    if len(vals) % 2:
                        nxt.append(vals[-1])
                    vals = nxt
                out_row[pl.ds(pl.multiple_of(c * _CB + g * 16, 16), 16)] = (
                    vals[0])
            return carry
        lax.fori_loop(0, _NG // 2, group_body, 0)

    def do_pair(k):
        p_lo = 32 * k
        f = p_lo // _D
        weighted = f == _F
        d = wid + 32 * (k % 2)
        tab = w_t if weighted else ebc_t
        pltpu.sync_copy(tab.at[0 if weighted else f, d], row_v)
        start_chunk(f, 0, 0, weighted)

        def step(i, carry):
            a = 2 * i
            wait_chunk(f, a, 0, weighted)
            start_chunk(f, a + 1, 1, weighted)
            accum_chunk(a, 0, weighted)
            wait_chunk(f, a + 1, 1, weighted)

            @pl.when(i < _NCH // 2 - 1)
            def _():
                start_chunk(f, a + 2, 0, weighted)
            accum_chunk(a + 1, 1, weighted)
            return carry
        lax.fori_loop(0, _NCH // 2, step, 0)
        pltpu.sync_copy(out_row, out_hbm.at[wid + 32 * k])

    for k in range(_PPW):
        do_pair(k)


@jax.jit
def _run(ebc_t, w_t, idx_all, wts):
    mesh = plsc.VectorSubcoreMesh(core_axis_name="c", subcore_axis_name="s")
    k = functools.partial(
        pl.kernel,
        mesh=mesh,
        out_type=jax.ShapeDtypeStruct((_NP, _B), jnp.float32),
        scratch_types=[
            pltpu.VMEM((_V,), jnp.float32),
            pltpu.VMEM((_L, _CB), jnp.int32),
            pltpu.VMEM((_L, _CB), jnp.int32),
            pltpu.VMEM((_L, _CB), jnp.float32),
            pltpu.VMEM((_L, _CB), jnp.float32),
            pltpu.VMEM((_B,), jnp.float32),
            pltpu.SemaphoreType.DMA,
            pltpu.SemaphoreType.DMA,
        ],
        compiler_params=pltpu.CompilerParams(needs_layout_passes=False),
    )(_body)
    return k(ebc_t, w_t, idx_all, wts)


def kernel(features_indices, weighted_features_indices,
           weighted_features_weights, ebc_tables, weighted_tables):
    ebc_t = ebc_tables.transpose(0, 2, 1)
    w_t = weighted_tables.transpose(0, 2, 1)
    it = features_indices.astype(jnp.int32).transpose(2, 1, 0)
    f0 = it[:, 0, :].reshape(_L, _NCH, _CB).transpose(1, 0, 2)
    f1 = it[:, 1, :].reshape(_L, _NCH, _CB).transpose(1, 0, 2)
    wi = weighted_features_indices.astype(jnp.int32).transpose(2, 1, 0)
    wi = wi[:, 0, :].reshape(_L, _NCH, _CB).transpose(1, 0, 2)
    idx_all = jnp.stack([f0, f1, wi])
    wts = weighted_features_weights.transpose(2, 1, 0)
    wts = wts[:, 0, :].reshape(_L, _NCH, _CB).transpose(1, 0, 2)
    out = _run(ebc_t, w_t, idx_all, wts)
    return out.T


# speedup vs baseline: 1.2236x; 1.2234x over previous
import functools

import jax
import jax.numpy as jnp
from jax import lax
from jax.experimental import pallas as pl
from jax.experimental.pallas import tpu as pltpu
from jax.experimental.pallas import tpu_sc as plsc

_B, _F, _FW, _L, _V, _D = 4096, 2, 1, 20, 100000, 64
_NF = _F + _FW
_NW = 32
_NP = _NF * _D
_PPW = _NP // _NW
_CB = 128
_NCH = _B // _CB
_NG = _CB // 16


def _body(ebc_t, w_t, idx_hbm, wts_hbm, out_hbm,
          row_v, idx_v0, idx_v1, wts_v0, wts_v1, out_row,
          sem_r, sem_a, sem_b):
    idx_bufs = (idx_v0, idx_v1)
    wts_bufs = (wts_v0, wts_v1)
    slot = lax.axis_index("s")
    sc = lax.axis_index("c")
    wid = slot * 2 + sc

    sems = (sem_a, sem_b)

    def start_chunk(f, c, b, weighted):
        pltpu.async_copy(idx_hbm.at[f, c], idx_bufs[b], sems[b])
        if weighted:
            pltpu.async_copy(wts_hbm.at[c], wts_bufs[b], sems[b])

    def wait_chunk(f, c, b, weighted):
        pltpu.make_async_copy(idx_hbm.at[f, c], idx_bufs[b], sems[b]).wait()
        if weighted:
            pltpu.make_async_copy(wts_hbm.at[c], wts_bufs[b], sems[b]).wait()

    def accum_chunk(c, b, weighted):
        def group_body(g2, carry):
            for u in range(2):
                g = 2 * g2 + u
                gl = pl.ds(pl.multiple_of(g * 16, 16), 16)
                ivs = [idx_bufs[b][l, gl] for l in range(_L)]
                vals = [plsc.load_gather(row_v, [iv]) for iv in ivs]
                if weighted:
                    ws = [wts_bufs[b][l, gl] for l in range(_L)]
                    vals = [v * w for v, w in zip(vals, ws)]
                while len(vals) > 1:
                    nxt = [vals[i] + vals[i + 1]
                           for i in range(0, len(vals) - 1, 2)]
                    if len(vals) % 2:
                        nxt.append(vals[-1])
                    vals = nxt
                out_row[pl.ds(pl.multiple_of(c * _CB + g * 16, 16), 16)] = (
                    vals[0])
            return carry
        lax.fori_loop(0, _NG // 2, group_body, 0)

    def do_pair(k):
        p_lo = 32 * k
        f = p_lo // _D
        weighted = f == _F
        d = wid + 32 * (k % 2)
        tab = w_t if weighted else ebc_t
        row_cp = pltpu.make_async_copy(tab.at[0 if weighted else f, d],
                                       row_v, sem_r)
        row_cp.start()
        start_chunk(f, 0, 0, weighted)
        start_chunk(f, 1, 1, weighted)
        row_cp.wait()

        def step(i, carry):
            a = 2 * i
            wait_chunk(f, a, 0, weighted)
            accum_chunk(a, 0, weighted)

            @pl.when(i < _NCH // 2 - 1)
            def _():
                start_chunk(f, a + 2, 0, weighted)
            wait_chunk(f, a + 1, 1, weighted)
            accum_chunk(a + 1, 1, weighted)

            @pl.when(i < _NCH // 2 - 1)
            def _():
                start_chunk(f, a + 3, 1, weighted)
            return carry
        lax.fori_loop(0, _NCH // 2, step, 0)
        pltpu.sync_copy(out_row, out_hbm.at[wid + 32 * k])

    for k in range(_PPW):
        do_pair(k)


@jax.jit
def _run(ebc_t, w_t, idx_all, wts):
    mesh = plsc.VectorSubcoreMesh(core_axis_name="c", subcore_axis_name="s")
    k = functools.partial(
        pl.kernel,
        mesh=mesh,
        out_type=jax.ShapeDtypeStruct((_NP, _B), jnp.float32),
        scratch_types=[
            pltpu.VMEM((_V,), jnp.float32),
            pltpu.VMEM((_L, _CB), jnp.int32),
            pltpu.VMEM((_L, _CB), jnp.int32),
            pltpu.VMEM((_L, _CB), jnp.float32),
            pltpu.VMEM((_L, _CB), jnp.float32),
            pltpu.VMEM((_B,), jnp.float32),
            pltpu.SemaphoreType.DMA,
            pltpu.SemaphoreType.DMA,
            pltpu.SemaphoreType.DMA,
        ],
        compiler_params=pltpu.CompilerParams(needs_layout_passes=False),
    )(_body)
    return k(ebc_t, w_t, idx_all, wts)


def kernel(features_indices, weighted_features_indices,
           weighted_features_weights, ebc_tables, weighted_tables):
    ebc_t = ebc_tables.transpose(0, 2, 1)
    w_t = weighted_tables.transpose(0, 2, 1)
    it = features_indices.astype(jnp.int32).transpose(2, 1, 0)
    f0 = it[:, 0, :].reshape(_L, _NCH, _CB).transpose(1, 0, 2)
    f1 = it[:, 1, :].reshape(_L, _NCH, _CB).transpose(1, 0, 2)
    wi = weighted_features_indices.astype(jnp.int32).transpose(2, 1, 0)
    wi = wi[:, 0, :].reshape(_L, _NCH, _CB).transpose(1, 0, 2)
    idx_all = jnp.stack([f0, f1, wi])
    wts = weighted_features_weights.transpose(2, 1, 0)
    wts = wts[:, 0, :].reshape(_L, _NCH, _CB).transpose(1, 0, 2)
    out = _run(ebc_t, w_t, idx_all, wts)
    return out.T


# speedup vs baseline: 1.2599x; 1.0296x over previous
import functools

import jax
import jax.numpy as jnp
from jax import lax
from jax.experimental import pallas as pl
from jax.experimental.pallas import tpu as pltpu
from jax.experimental.pallas import tpu_sc as plsc

_B, _F, _FW, _L, _V, _D = 4096, 2, 1, 20, 100000, 64
_NF = _F + _FW
_NW = 32
_NP = _NF * _D
_PPW = _NP // _NW
_CB = 128
_NCH = _B // _CB
_NG = _CB // 16


def _body(ebc_t, w_t, idx_hbm, wts_hbm, out_hbm,
          row_v, idx_v0, idx_v1, wts_v0, wts_v1, out_row,
          sem_r, sem_a, sem_b):
    idx_bufs = (idx_v0, idx_v1)
    wts_bufs = (wts_v0, wts_v1)
    slot = lax.axis_index("s")
    sc = lax.axis_index("c")
    wid = slot * 2 + sc

    sems = (sem_a, sem_b)

    def start_chunk(f, c, b, weighted):
        pltpu.async_copy(idx_hbm.at[f, c], idx_bufs[b], sems[b])
        if weighted:
            pltpu.async_copy(wts_hbm.at[c], wts_bufs[b], sems[b])

    def wait_chunk(f, c, b, weighted):
        pltpu.make_async_copy(idx_hbm.at[f, c], idx_bufs[b], sems[b]).wait()
        if weighted:
            pltpu.make_async_copy(wts_hbm.at[c], wts_bufs[b], sems[b]).wait()

    def accum_chunk(c, b, weighted):
        @plsc.parallel_loop(0, _NG, unroll=2)
        def _(g):
            gl = pl.ds(pl.multiple_of(g * 16, 16), 16)
            ivs = [idx_bufs[b][l, gl] for l in range(_L)]
            vals = [plsc.load_gather(row_v, [iv]) for iv in ivs]
            if weighted:
                ws = [wts_bufs[b][l, gl] for l in range(_L)]
                vals = [v * w for v, w in zip(vals, ws)]
            while len(vals) > 1:
                nxt = [vals[i] + vals[i + 1]
                       for i in range(0, len(vals) - 1, 2)]
                if len(vals) % 2:
                    nxt.append(vals[-1])
                vals = nxt
            out_row[pl.ds(pl.multiple_of(c * _CB + g * 16, 16), 16)] = vals[0]

    def do_pair(k):
        p_lo = 32 * k
        f = p_lo // _D
        weighted = f == _F
        d = wid + 32 * (k % 2)
        tab = w_t if weighted else ebc_t
        row_cp = pltpu.make_async_copy(tab.at[0 if weighted else f, d],
                                       row_v, sem_r)
        row_cp.start()
        start_chunk(f, 0, 0, weighted)
        start_chunk(f, 1, 1, weighted)
        row_cp.wait()

        def step(i, carry):
            a = 2 * i
            wait_chunk(f, a, 0, weighted)
            accum_chunk(a, 0, weighted)

            @pl.when(i < _NCH // 2 - 1)
            def _():
                start_chunk(f, a + 2, 0, weighted)
            wait_chunk(f, a + 1, 1, weighted)
            accum_chunk(a + 1, 1, weighted)

            @pl.when(i < _NCH // 2 - 1)
            def _():
                start_chunk(f, a + 3, 1, weighted)
            return carry
        lax.fori_loop(0, _NCH // 2, step, 0)
        pltpu.sync_copy(out_row, out_hbm.at[wid + 32 * k])

    for k in range(_PPW):
        do_pair(k)


@jax.jit
def _run(ebc_t, w_t, idx_all, wts):
    mesh = plsc.VectorSubcoreMesh(core_axis_name="c", subcore_axis_name="s")
    k = functools.partial(
        pl.kernel,
        mesh=mesh,
        out_type=jax.ShapeDtypeStruct((_NP, _B), jnp.float32),
        scratch_types=[
            pltpu.VMEM((_V,), jnp.float32),
            pltpu.VMEM((_L, _CB), jnp.int32),
            pltpu.VMEM((_L, _CB), jnp.int32),
            pltpu.VMEM((_L, _CB), jnp.float32),
            pltpu.VMEM((_L, _CB), jnp.float32),
            pltpu.VMEM((_B,), jnp.float32),
            pltpu.SemaphoreType.DMA,
            pltpu.SemaphoreType.DMA,
            pltpu.SemaphoreType.DMA,
        ],
        compiler_params=pltpu.CompilerParams(needs_layout_passes=False),
    )(_body)
    return k(ebc_t, w_t, idx_all, wts)


def kernel(features_indices, weighted_features_indices,
           weighted_features_weights, ebc_tables, weighted_tables):
    ebc_t = ebc_tables.transpose(0, 2, 1)
    w_t = weighted_tables.transpose(0, 2, 1)
    it = features_indices.astype(jnp.int32).transpose(2, 1, 0)
    f0 = it[:, 0, :].reshape(_L, _NCH, _CB).transpose(1, 0, 2)
    f1 = it[:, 1, :].reshape(_L, _NCH, _CB).transpose(1, 0, 2)
    wi = weighted_features_indices.astype(jnp.int32).transpose(2, 1, 0)
    wi = wi[:, 0, :].reshape(_L, _NCH, _CB).transpose(1, 0, 2)
    idx_all = jnp.stack([f0, f1, wi])
    wts = weighted_features_weights.transpose(2, 1, 0)
    wts = wts[:, 0, :].reshape(_L, _NCH, _CB).transpose(1, 0, 2)
    out = _run(ebc_t, w_t, idx_all, wts)
    return out.T
